# Initial kernel scaffold; baseline (speedup 1.0000x reference)
import jax
import jax.numpy as jnp
from jax.experimental import pallas as pl


def kernel(x_nchw, w1, b1, w2, b2):
    raise NotImplementedError("write your pallas kernel here")


# speedup vs baseline: 2.2132x; 2.2132x over previous
import jax
import jax.numpy as jnp
from jax.experimental import pallas as pl
from jax.experimental.pallas import tpu as pltpu


def kernel(x_nchw, w1, b1, w2, b2):
    n, c, h, w = x_nchw.shape
    s = h * w
    hidden = w1.shape[1]
    inv_s = 1.0 / float(s)

    x3 = x_nchw.reshape(n, c, s)
    b1c = b1.reshape(hidden, 1)
    b2c = b2.reshape(1, 1)

    def _body(x_ref, w1_ref, b1_ref, w2_ref, b2_ref, o_ref):
        tile = x_ref[0]
        pooled = jnp.sum(tile, axis=1, keepdims=True) * inv_s
        hid = jax.lax.dot_general(
            w1_ref[...], pooled, (((0,), (0,)), ((), ())),
            preferred_element_type=jnp.float32)
        hid = jnp.maximum(hid + b1_ref[...], 0.0)
        logit = jax.lax.dot_general(
            w2_ref[...], hid, (((0,), (0,)), ((), ())),
            preferred_element_type=jnp.float32) + b2_ref[...]
        o_ref[...] = jax.nn.sigmoid(logit).reshape(1, 1, 1)

    out = pl.pallas_call(
        _body,
        out_shape=jax.ShapeDtypeStruct((n, 1, 1), jnp.float32),
        grid=(n,),
        in_specs=[
            pl.BlockSpec((1, c, s), lambda i: (i, 0, 0)),
            pl.BlockSpec((c, hidden), lambda i: (0, 0)),
            pl.BlockSpec((hidden, 1), lambda i: (0, 0)),
            pl.BlockSpec((hidden, 1), lambda i: (0, 0)),
            pl.BlockSpec((1, 1), lambda i: (0, 0)),
        ],
        out_specs=pl.BlockSpec((1, 1, 1), lambda i: (i, 0, 0)),
        compiler_params=pltpu.CompilerParams(
            dimension_semantics=("parallel",),
            vmem_limit_bytes=64 * 1024 * 1024,
        ),
    )(x3, w1, b1c, w2, b2c)
    return out.reshape(n, 1)
